# Initial kernel scaffold; baseline (speedup 1.0000x reference)
import jax
import jax.numpy as jnp
from jax.experimental import pallas as pl


def kernel(x, edge_index, edge_weight, W1, b1, W2, b2):
    raise NotImplementedError("write your pallas kernel here")


# speedup vs baseline: 8.2536x; 8.2536x over previous
import functools

import jax
import jax.numpy as jnp
from jax import lax
from jax.experimental import pallas as pl
from jax.experimental.pallas import tpu as pltpu
from jax.experimental.pallas import tpu_sc as plsc

N = 10000
E = 320000
D = 128

NC = 2
NS = 16
NW = NC * NS
CH = 80
EP = 327680
BLK = 32
ROWS_PT = EP // CH // NW
NBLK = ROWS_PT // BLK

NR_MAIN = 632
NR_TAIL = N - 15 * NR_MAIN

NP_DEG = 10240

_mesh = plsc.VectorSubcoreMesh(core_axis_name="c", subcore_axis_name="s")

_GATHER_DNUMS = lax.GatherDimensionNumbers(
    offset_dims=(), collapsed_slice_dims=(0,), start_index_map=(0,))


def _bcast_lane(v16, i):
    idx = jnp.full((16, 1), i, jnp.int32)
    return lax.gather(v16, idx, _GATHER_DNUMS, slice_sizes=(1,),
                      mode=lax.GatherScatterMode.PROMISE_IN_BOUNDS)


@functools.partial(
    pl.kernel, mesh=_mesh,
    out_type=jax.ShapeDtypeStruct((NC, NP_DEG), jnp.float32),
    scratch_types=[
        pltpu.VMEM((BLK, CH), jnp.int32),
        pltpu.VMEM((BLK, CH), jnp.float32),
        pltpu.VMEM((640,), jnp.float32),
        pltpu.VMEM_SHARED((NP_DEG,), jnp.float32),
    ])
def _deg_kernel(col_hbm, ew_hbm, out_hbm, col_v, ew_v, zb, acc):
    c = lax.axis_index("c")
    s = lax.axis_index("s")
    wid = c * NS + s

    def zloop(i, carry):
        zb[pl.ds(i * 16, 16)] = jnp.zeros((16,), jnp.float32)
        return carry
    lax.fori_loop(0, 40, zloop, 0)

    pltpu.sync_copy(zb, acc.at[pl.ds(s * 640, 640)])

    plsc.subcore_barrier()

    def block(b, carry):
        r0 = wid * ROWS_PT + b * BLK
        pltpu.sync_copy(col_hbm.at[pl.ds(r0, BLK)], col_v)
        pltpu.sync_copy(ew_hbm.at[pl.ds(r0, BLK)], ew_v)

        def chunk(k, inner):
            pltpu.sync_copy(ew_v.at[k], acc.at[col_v.at[k]], add=True)
            return inner
        return lax.fori_loop(0, BLK, chunk, carry)
    lax.fori_loop(0, NBLK, block, 0)

    plsc.subcore_barrier()
    pltpu.sync_copy(acc.at[pl.ds(s * 640, 640)], out_hbm.at[c, pl.ds(s * 640, 640)])


@functools.partial(
    pl.kernel, mesh=_mesh,
    out_type=jax.ShapeDtypeStruct((NC, N, D), jnp.float32),
    scratch_types=[
        pltpu.VMEM((BLK, CH), jnp.int32),
        pltpu.VMEM((BLK, CH), jnp.int32),
        pltpu.VMEM((BLK, CH), jnp.float32),
        pltpu.VMEM((CH, D), jnp.float32),
        pltpu.VMEM_SHARED((N, D), jnp.float32),
        pltpu.SemaphoreType.DMA,
    ])
def _agg_kernel(h_hbm, row_hbm, col_hbm, ew_hbm, out_hbm,
                row_v, col_v, ew_v, msg, acc, sem):
    c = lax.axis_index("c")
    s = lax.axis_index("s")
    wid = c * NS + s

    def zrow(i, carry):
        for j in range(D // 16):
            msg[i, pl.ds(j * 16, 16)] = jnp.zeros((16,), jnp.float32)
        return carry
    lax.fori_loop(0, CH, zrow, 0)

    base = s * NR_MAIN

    @pl.when(s < 15)
    def _():
        for q in range(7):
            pltpu.sync_copy(msg, acc.at[pl.ds(base + q * CH, CH)])
        pltpu.sync_copy(msg.at[pl.ds(0, NR_MAIN - 7 * CH)],
                        acc.at[pl.ds(base + 7 * CH, NR_MAIN - 7 * CH)])

    @pl.when(s == 15)
    def _():
        for q in range(6):
            pltpu.sync_copy(msg, acc.at[pl.ds(15 * NR_MAIN + q * CH, CH)])
        pltpu.sync_copy(msg.at[pl.ds(0, NR_TAIL - 6 * CH)],
                        acc.at[pl.ds(15 * NR_MAIN + 6 * CH, NR_TAIL - 6 * CH)])

    plsc.subcore_barrier()

    def block(b, carry):
        r0 = wid * ROWS_PT + b * BLK
        pltpu.sync_copy(row_hbm.at[pl.ds(r0, BLK)], row_v)
        pltpu.sync_copy(col_hbm.at[pl.ds(r0, BLK)], col_v)
        pltpu.sync_copy(ew_hbm.at[pl.ds(r0, BLK)], ew_v)

        def chunk(k, inner):
            pltpu.async_copy(h_hbm.at[row_v.at[k]], msg, sem).wait()

            def scale16(g, inner2):
                ew16 = ew_v[k, pl.ds(g * 16, 16)]
                for i in range(16):
                    nb = _bcast_lane(ew16, i)
                    e = g * 16 + i
                    for j in range(D // 16):
                        msg[e, pl.ds(j * 16, 16)] = msg[e, pl.ds(j * 16, 16)] * nb
                return inner2
            lax.fori_loop(0, CH // 16, scale16, 0)

            pltpu.sync_copy(msg, acc.at[col_v.at[k]], add=True)
            return inner
        return lax.fori_loop(0, BLK, chunk, carry)
    lax.fori_loop(0, NBLK, block, 0)

    plsc.subcore_barrier()

    @pl.when(s < 15)
    def _():
        pltpu.sync_copy(acc.at[pl.ds(base, NR_MAIN)],
                        out_hbm.at[c, pl.ds(base, NR_MAIN)])

    @pl.when(s == 15)
    def _():
        pltpu.sync_copy(acc.at[pl.ds(15 * NR_MAIN, NR_TAIL)],
                        out_hbm.at[c, pl.ds(15 * NR_MAIN, NR_TAIL)])


RB = 1024
GRID = 10


def _dis_block(pdeg_ref, i):
    deg = pdeg_ref[0, pl.ds(i * RB, RB)] + pdeg_ref[1, pl.ds(i * RB, RB)] + 1.0
    return lax.rsqrt(deg)


def _mm1_body(pdeg_ref, x_ref, w_ref, o_ref):
    i = pl.program_id(0)
    dis = _dis_block(pdeg_ref, i)
    o_ref[...] = jnp.dot(x_ref[...], w_ref[...],
                         preferred_element_type=jnp.float32) * dis[:, None]


def _mid_body(pdeg_ref, p_ref, ht_ref, w_ref, b_ref, o_ref):
    i = pl.program_id(0)
    dis = _dis_block(pdeg_ref, i)
    agg = p_ref[0] + p_ref[1] + ht_ref[...]
    h1 = jnp.maximum(agg * dis[:, None] + b_ref[...][None, :], 0.0)
    o_ref[...] = jnp.dot(h1, w_ref[...],
                         preferred_element_type=jnp.float32) * dis[:, None]


def _fin_body(pdeg_ref, p_ref, ht_ref, b_ref, o_ref):
    i = pl.program_id(0)
    dis = _dis_block(pdeg_ref, i)
    agg = p_ref[0] + p_ref[1] + ht_ref[...]
    o_ref[...] = jax.nn.sigmoid(agg * dis[:, None] + b_ref[...][None, :])


_full_pdeg = pl.BlockSpec((NC, NP_DEG), lambda i: (0, 0))
_rows = pl.BlockSpec((RB, D), lambda i: (i, 0))
_part = pl.BlockSpec((NC, RB, D), lambda i: (0, i, 0))
_wspec = pl.BlockSpec((D, D), lambda i: (0, 0))
_bspec = pl.BlockSpec((D,), lambda i: (0,))
_out_sds = jax.ShapeDtypeStruct((N, D), jnp.float32)

_mm1 = pl.pallas_call(
    _mm1_body, grid=(GRID,),
    in_specs=[_full_pdeg, _rows, _wspec],
    out_specs=_rows, out_shape=_out_sds)

_mid = pl.pallas_call(
    _mid_body, grid=(GRID,),
    in_specs=[_full_pdeg, _part, _rows, _wspec, _bspec],
    out_specs=_rows, out_shape=_out_sds)

_fin = pl.pallas_call(
    _fin_body, grid=(GRID,),
    in_specs=[_full_pdeg, _part, _rows, _bspec],
    out_specs=_rows, out_shape=_out_sds)


def kernel(x, edge_index, edge_weight, W1, b1, W2, b2):
    pad = EP - E
    ei = edge_index.astype(jnp.int32)
    row = jnp.concatenate([ei[0], jnp.zeros((pad,), jnp.int32)]).reshape(EP // CH, CH)
    col = jnp.concatenate([ei[1], jnp.zeros((pad,), jnp.int32)]).reshape(EP // CH, CH)
    ew = jnp.concatenate([edge_weight, jnp.zeros((pad,), jnp.float32)]).reshape(EP // CH, CH)

    pdeg = _deg_kernel(col, ew)
    h1t = _mm1(pdeg, x, W1)
    p1 = _agg_kernel(h1t, row, col, ew)
    h2t = _mid(pdeg, p1, h1t, W2, b1)
    p2 = _agg_kernel(h2t, row, col, ew)
    return _fin(pdeg, p2, h2t, b2)


# speedup vs baseline: 10.5978x; 1.2840x over previous
import functools

import jax
import jax.numpy as jnp
from jax import lax
from jax.experimental import pallas as pl
from jax.experimental.pallas import tpu as pltpu
from jax.experimental.pallas import tpu_sc as plsc

N = 10000
E = 320000
D = 128

NC = 2
NS = 16
NW = NC * NS
CH = 128
EP = 327680
NCH = EP // CH // NW
BI = 8
NBLK = NCH // BI

NR_MAIN = 632
NR_TAIL = N - 15 * NR_MAIN

NP_DEG = 10240

_mesh = plsc.VectorSubcoreMesh(core_axis_name="c", subcore_axis_name="s")

_GATHER_DNUMS = lax.GatherDimensionNumbers(
    offset_dims=(), collapsed_slice_dims=(0,), start_index_map=(0,))


def _bcast_lane(v16, i):
    idx = jnp.full((16, 1), i, jnp.int32)
    return lax.gather(v16, idx, _GATHER_DNUMS, slice_sizes=(1,),
                      mode=lax.GatherScatterMode.PROMISE_IN_BOUNDS)


@functools.partial(
    pl.kernel, mesh=_mesh,
    out_type=jax.ShapeDtypeStruct((NC, NP_DEG), jnp.float32),
    scratch_types=[
        pltpu.VMEM((BI, CH), jnp.int32),
        pltpu.VMEM((BI, CH), jnp.float32),
        pltpu.VMEM((640,), jnp.float32),
        pltpu.VMEM_SHARED((NP_DEG,), jnp.float32),
    ])
def _deg_kernel(col_hbm, ew_hbm, out_hbm, col_v, ew_v, zb, acc):
    c = lax.axis_index("c")
    s = lax.axis_index("s")
    wid = c * NS + s

    def zloop(i, carry):
        zb[pl.ds(i * 16, 16)] = jnp.zeros((16,), jnp.float32)
        return carry
    lax.fori_loop(0, 40, zloop, 0)

    pltpu.sync_copy(zb, acc.at[pl.ds(s * 640, 640)])

    plsc.subcore_barrier()

    def block(b, carry):
        r0 = wid * NCH + b * BI
        pltpu.sync_copy(col_hbm.at[pl.ds(r0, BI)], col_v)
        pltpu.sync_copy(ew_hbm.at[pl.ds(r0, BI)], ew_v)

        def chunk(k, inner):
            pltpu.sync_copy(ew_v.at[k], acc.at[col_v.at[k]], add=True)
            return inner
        return lax.fori_loop(0, BI, chunk, carry)
    lax.fori_loop(0, NBLK, block, 0)

    plsc.subcore_barrier()
    pltpu.sync_copy(acc.at[pl.ds(s * 640, 640)], out_hbm.at[c, pl.ds(s * 640, 640)])


@functools.partial(
    pl.kernel, mesh=_mesh,
    out_type=jax.ShapeDtypeStruct((NC, N, D), jnp.float32),
    scratch_types=[
        pltpu.VMEM((2, BI, CH), jnp.int32),
        pltpu.VMEM((2, BI, CH), jnp.int32),
        pltpu.VMEM((2, BI, CH), jnp.float32),
        pltpu.VMEM((CH, D), jnp.float32),
        pltpu.VMEM((CH, D), jnp.float32),
        pltpu.VMEM_SHARED((N, D), jnp.float32),
        pltpu.SemaphoreType.DMA,
        pltpu.SemaphoreType.DMA,
        pltpu.SemaphoreType.DMA,
        pltpu.SemaphoreType.DMA,
        pltpu.SemaphoreType.DMA,
        pltpu.SemaphoreType.DMA,
    ])
def _agg_kernel(h_hbm, row_hbm, col_hbm, ew_hbm, out_hbm,
                rowb, colb, ewb, m0, m1, acc, g0, g1, s0, s1, t0, t1):
    c = lax.axis_index("c")
    s = lax.axis_index("s")
    wid = c * NS + s
    msgs = (m0, m1)
    gsem = (g0, g1)
    ssem = (s0, s1)
    tsem = (t0, t1)
    r0 = wid * NCH

    def stage(j, p, sync=False):
        copy = pltpu.sync_copy if sync else (
            lambda src, dst: pltpu.async_copy(src, dst, tsem[p]))
        copy(row_hbm.at[pl.ds(r0 + j * BI, BI)], rowb.at[p])
        copy(col_hbm.at[pl.ds(r0 + j * BI, BI)], colb.at[p])
        copy(ew_hbm.at[pl.ds(r0 + j * BI, BI)], ewb.at[p])

    def wait_stage(j, p):
        pltpu.make_async_copy(row_hbm.at[pl.ds(r0 + j * BI, BI)], rowb.at[p], tsem[p]).wait()
        pltpu.make_async_copy(col_hbm.at[pl.ds(r0 + j * BI, BI)], colb.at[p], tsem[p]).wait()
        pltpu.make_async_copy(ew_hbm.at[pl.ds(r0 + j * BI, BI)], ewb.at[p], tsem[p]).wait()

    def ig(p, kk, b):
        pltpu.async_copy(h_hbm.at[rowb.at[p, kk]], msgs[b], gsem[b])

    def wg(p, kk, b):
        pltpu.make_async_copy(h_hbm.at[rowb.at[p, kk]], msgs[b], gsem[b]).wait()

    def isc(p, kk, b):
        pltpu.async_copy(msgs[b], acc.at[colb.at[p, kk]], ssem[b], add=True)

    def wsc(p, kk, b):
        pltpu.make_async_copy(msgs[b], acc.at[colb.at[p, kk]], ssem[b]).wait()

    def scale(p, kk, b):
        mb = msgs[b]

        def scale16(g, carry):
            ew16 = ewb[p, kk, pl.ds(g * 16, 16)]
            for i in range(16):
                nb = _bcast_lane(ew16, i)
                e = g * 16 + i
                for jj in range(D // 16):
                    mb[e, pl.ds(jj * 16, 16)] = mb[e, pl.ds(jj * 16, 16)] * nb
            return carry
        lax.fori_loop(0, CH // 16, scale16, 0)

    def zrow(i, carry):
        for j in range(D // 16):
            m0[i, pl.ds(j * 16, 16)] = jnp.zeros((16,), jnp.float32)
        return carry
    lax.fori_loop(0, CH, zrow, 0)

    base = s * NR_MAIN

    @pl.when(s < 15)
    def _():
        for q in range(NR_MAIN // CH):
            pltpu.sync_copy(m0, acc.at[pl.ds(base + q * CH, CH)])
        rem = NR_MAIN - (NR_MAIN // CH) * CH
        pltpu.sync_copy(m0.at[pl.ds(0, rem)],
                        acc.at[pl.ds(base + NR_MAIN - rem, rem)])

    @pl.when(s == 15)
    def _():
        for q in range(NR_TAIL // CH):
            pltpu.sync_copy(m0, acc.at[pl.ds(15 * NR_MAIN + q * CH, CH)])
        rem = NR_TAIL - (NR_TAIL // CH) * CH
        pltpu.sync_copy(m0.at[pl.ds(0, rem)],
                        acc.at[pl.ds(15 * NR_MAIN + NR_TAIL - rem, rem)])

    plsc.subcore_barrier()

    def block_ops(j, p, is_first, is_last):
        for kk in range(BI):
            b = kk % 2
            if kk == 0:
                if not is_first:
                    wsc(p ^ 1, BI - 1, 1)
                if not is_last:
                    stage(j + 1, p ^ 1)
            else:
                wsc(p, kk - 1, (kk - 1) % 2)
            if kk == BI - 1:
                if not is_last:
                    wait_stage(j + 1, p ^ 1)
                    ig(p ^ 1, 0, 0)
            else:
                ig(p, kk + 1, (kk + 1) % 2)
            wg(p, kk, b)
            scale(p, kk, b)
            isc(p, kk, b)

    stage(0, 0, sync=True)
    ig(0, 0, 0)
    block_ops(0, 0, is_first=True, is_last=False)

    def two_blocks(i, carry):
        block_ops(1 + 2 * i, 1, False, False)
        block_ops(2 + 2 * i, 0, False, False)
        return carry
    lax.fori_loop(0, (NBLK - 2) // 2, two_blocks, 0)

    block_ops(NBLK - 1, (NBLK - 1) % 2, is_first=False, is_last=True)
    wsc((NBLK - 1) % 2, BI - 1, 1)

    plsc.subcore_barrier()

    @pl.when(s < 15)
    def _():
        pltpu.sync_copy(acc.at[pl.ds(base, NR_MAIN)],
                        out_hbm.at[c, pl.ds(base, NR_MAIN)])

    @pl.when(s == 15)
    def _():
        pltpu.sync_copy(acc.at[pl.ds(15 * NR_MAIN, NR_TAIL)],
                        out_hbm.at[c, pl.ds(15 * NR_MAIN, NR_TAIL)])


RB = 1024
GRID = 10


def _dis_block(pdeg_ref, i):
    deg = pdeg_ref[0, pl.ds(i * RB, RB)] + pdeg_ref[1, pl.ds(i * RB, RB)] + 1.0
    return lax.rsqrt(deg)


def _mm1_body(pdeg_ref, x_ref, w_ref, o_ref):
    i = pl.program_id(0)
    dis = _dis_block(pdeg_ref, i)
    o_ref[...] = jnp.dot(x_ref[...], w_ref[...],
                         preferred_element_type=jnp.float32) * dis[:, None]


def _mid_body(pdeg_ref, p_ref, ht_ref, w_ref, b_ref, o_ref):
    i = pl.program_id(0)
    dis = _dis_block(pdeg_ref, i)
    agg = p_ref[0] + p_ref[1] + ht_ref[...]
    h1 = jnp.maximum(agg * dis[:, None] + b_ref[...][None, :], 0.0)
    o_ref[...] = jnp.dot(h1, w_ref[...],
                         preferred_element_type=jnp.float32) * dis[:, None]


def _fin_body(pdeg_ref, p_ref, ht_ref, b_ref, o_ref):
    i = pl.program_id(0)
    dis = _dis_block(pdeg_ref, i)
    agg = p_ref[0] + p_ref[1] + ht_ref[...]
    o_ref[...] = jax.nn.sigmoid(agg * dis[:, None] + b_ref[...][None, :])


_full_pdeg = pl.BlockSpec((NC, NP_DEG), lambda i: (0, 0))
_rows = pl.BlockSpec((RB, D), lambda i: (i, 0))
_part = pl.BlockSpec((NC, RB, D), lambda i: (0, i, 0))
_wspec = pl.BlockSpec((D, D), lambda i: (0, 0))
_bspec = pl.BlockSpec((D,), lambda i: (0,))
_out_sds = jax.ShapeDtypeStruct((N, D), jnp.float32)

_mm1 = pl.pallas_call(
    _mm1_body, grid=(GRID,),
    in_specs=[_full_pdeg, _rows, _wspec],
    out_specs=_rows, out_shape=_out_sds)

_mid = pl.pallas_call(
    _mid_body, grid=(GRID,),
    in_specs=[_full_pdeg, _part, _rows, _wspec, _bspec],
    out_specs=_rows, out_shape=_out_sds)

_fin = pl.pallas_call(
    _fin_body, grid=(GRID,),
    in_specs=[_full_pdeg, _part, _rows, _bspec],
    out_specs=_rows, out_shape=_out_sds)


def kernel(x, edge_index, edge_weight, W1, b1, W2, b2):
    pad = EP - E
    ei = edge_index.astype(jnp.int32)
    row = jnp.concatenate([ei[0], jnp.zeros((pad,), jnp.int32)]).reshape(EP // CH, CH)
    col = jnp.concatenate([ei[1], jnp.zeros((pad,), jnp.int32)]).reshape(EP // CH, CH)
    ew = jnp.concatenate([edge_weight, jnp.zeros((pad,), jnp.float32)]).reshape(EP // CH, CH)

    pdeg = _deg_kernel(col, ew)
    h1t = _mm1(pdeg, x, W1)
    p1 = _agg_kernel(h1t, row, col, ew)
    h2t = _mid(pdeg, p1, h1t, W2, b1)
    p2 = _agg_kernel(h2t, row, col, ew)
    return _fin(pdeg, p2, h2t, b2)


# speedup vs baseline: 27.5404x; 2.5987x over previous
import functools

import jax
import jax.numpy as jnp
from jax import lax
from jax.experimental import pallas as pl
from jax.experimental.pallas import tpu as pltpu
from jax.experimental.pallas import tpu_sc as plsc

N = 10000
E = 320000
D = 128

NC = 2
NS = 16
NW = NC * NS
CH = 128
EP = 327680
NCH = EP // CH // NW
BI = 8
NBLK = NCH // BI

NR_MAIN = 632
NR_TAIL = N - 15 * NR_MAIN

NP_DEG = 10240

_mesh = plsc.VectorSubcoreMesh(core_axis_name="c", subcore_axis_name="s")

_GATHER_DNUMS = lax.GatherDimensionNumbers(
    offset_dims=(), collapsed_slice_dims=(0,), start_index_map=(0,))


def _bcast_lane(v16, i):
    idx = jnp.full((16, 1), i, jnp.int32)
    return lax.gather(v16, idx, _GATHER_DNUMS, slice_sizes=(1,),
                      mode=lax.GatherScatterMode.PROMISE_IN_BOUNDS)


@functools.partial(
    pl.kernel, mesh=_mesh,
    out_type=jax.ShapeDtypeStruct((NC, NP_DEG), jnp.float32),
    scratch_types=[
        pltpu.VMEM((BI, CH), jnp.int32),
        pltpu.VMEM((BI, CH), jnp.float32),
        pltpu.VMEM((640,), jnp.float32),
        pltpu.VMEM_SHARED((NP_DEG,), jnp.float32),
    ])
def _deg_kernel(col_hbm, ew_hbm, out_hbm, col_v, ew_v, zb, acc):
    c = lax.axis_index("c")
    s = lax.axis_index("s")
    wid = c * NS + s

    def zloop(i, carry):
        zb[pl.ds(i * 16, 16)] = jnp.zeros((16,), jnp.float32)
        return carry
    lax.fori_loop(0, 40, zloop, 0)

    pltpu.sync_copy(zb, acc.at[pl.ds(s * 640, 640)])

    plsc.subcore_barrier()

    def block(b, carry):
        r0 = wid * NCH + b * BI
        pltpu.sync_copy(col_hbm.at[pl.ds(r0, BI)], col_v)
        pltpu.sync_copy(ew_hbm.at[pl.ds(r0, BI)], ew_v)

        def chunk(k, inner):
            pltpu.sync_copy(ew_v.at[k], acc.at[col_v.at[k]], add=True)
            return inner
        return lax.fori_loop(0, BI, chunk, carry)
    lax.fori_loop(0, NBLK, block, 0)

    plsc.subcore_barrier()
    pltpu.sync_copy(acc.at[pl.ds(s * 640, 640)], out_hbm.at[c, pl.ds(s * 640, 640)])


@functools.partial(
    pl.kernel, mesh=_mesh,
    out_type=jax.ShapeDtypeStruct((NC, N, D), jnp.float32),
    scratch_types=[
        pltpu.VMEM((2, BI, CH), jnp.int32),
        pltpu.VMEM((2, BI, CH), jnp.int32),
        pltpu.VMEM((2, BI, CH), jnp.float32),
        pltpu.VMEM((CH, D), jnp.float32),
        pltpu.VMEM((CH, D), jnp.float32),
        pltpu.VMEM_SHARED((N, D), jnp.float32),
        pltpu.SemaphoreType.DMA,
        pltpu.SemaphoreType.DMA,
        pltpu.SemaphoreType.DMA,
        pltpu.SemaphoreType.DMA,
        pltpu.SemaphoreType.DMA,
        pltpu.SemaphoreType.DMA,
    ])
def _agg_kernel(h_hbm, row_hbm, col_hbm, ew_hbm, out_hbm,
                rowb, colb, ewb, m0, m1, acc, g0, g1, s0, s1, t0, t1):
    c = lax.axis_index("c")
    s = lax.axis_index("s")
    wid = c * NS + s
    msgs = (m0, m1)
    gsem = (g0, g1)
    ssem = (s0, s1)
    tsem = (t0, t1)
    r0 = wid * NCH

    def stage(j, p, sync=False):
        copy = pltpu.sync_copy if sync else (
            lambda src, dst: pltpu.async_copy(src, dst, tsem[p]))
        copy(row_hbm.at[pl.ds(r0 + j * BI, BI)], rowb.at[p])
        copy(col_hbm.at[pl.ds(r0 + j * BI, BI)], colb.at[p])
        copy(ew_hbm.at[pl.ds(r0 + j * BI, BI)], ewb.at[p])

    def wait_stage(j, p):
        pltpu.make_async_copy(row_hbm.at[pl.ds(r0 + j * BI, BI)], rowb.at[p], tsem[p]).wait()
        pltpu.make_async_copy(col_hbm.at[pl.ds(r0 + j * BI, BI)], colb.at[p], tsem[p]).wait()
        pltpu.make_async_copy(ew_hbm.at[pl.ds(r0 + j * BI, BI)], ewb.at[p], tsem[p]).wait()

    def ig(p, kk, b):
        pltpu.async_copy(h_hbm.at[rowb.at[p, kk]], msgs[b], gsem[b])

    def wg(p, kk, b):
        pltpu.make_async_copy(h_hbm.at[rowb.at[p, kk]], msgs[b], gsem[b]).wait()

    def isc(p, kk, b):
        pltpu.async_copy(msgs[b], acc.at[colb.at[p, kk]], ssem[b], add=True)

    def wsc(p, kk, b):
        pltpu.make_async_copy(msgs[b], acc.at[colb.at[p, kk]], ssem[b]).wait()

    def scale(p, kk, b):
        mb = msgs[b]

        def scale16(g, carry):
            ew16 = ewb[p, kk, pl.ds(g * 16, 16)]
            for i in range(16):
                nb = _bcast_lane(ew16, i)
                e = g * 16 + i
                for jj in range(D // 16):
                    mb[e, pl.ds(jj * 16, 16)] = mb[e, pl.ds(jj * 16, 16)] * nb
            return carry
        lax.fori_loop(0, CH // 16, scale16, 0)

    def zrow(i, carry):
        for j in range(D // 16):
            m0[i, pl.ds(j * 16, 16)] = jnp.zeros((16,), jnp.float32)
        return carry
    lax.fori_loop(0, CH, zrow, 0)

    base = s * NR_MAIN

    @pl.when(s < 15)
    def _():
        for q in range(NR_MAIN // CH):
            pltpu.sync_copy(m0, acc.at[pl.ds(base + q * CH, CH)])
        rem = NR_MAIN - (NR_MAIN // CH) * CH
        pltpu.sync_copy(m0.at[pl.ds(0, rem)],
                        acc.at[pl.ds(base + NR_MAIN - rem, rem)])

    @pl.when(s == 15)
    def _():
        for q in range(NR_TAIL // CH):
            pltpu.sync_copy(m0, acc.at[pl.ds(15 * NR_MAIN + q * CH, CH)])
        rem = NR_TAIL - (NR_TAIL // CH) * CH
        pltpu.sync_copy(m0.at[pl.ds(0, rem)],
                        acc.at[pl.ds(15 * NR_MAIN + NR_TAIL - rem, rem)])

    plsc.subcore_barrier()

    def block_ops(j, p, is_first, is_last):
        for kk in range(BI):
            b = kk % 2
            if kk == 0:
                if not is_first:
                    wsc(p ^ 1, BI - 1, 1)
                if not is_last:
                    stage(j + 1, p ^ 1)
            else:
                wsc(p, kk - 1, (kk - 1) % 2)
            if kk == BI - 1:
                if not is_last:
                    wait_stage(j + 1, p ^ 1)
                    ig(p ^ 1, 0, 0)
            else:
                ig(p, kk + 1, (kk + 1) % 2)
            wg(p, kk, b)
            scale(p, kk, b)
            isc(p, kk, b)

    stage(0, 0, sync=True)
    ig(0, 0, 0)
    block_ops(0, 0, is_first=True, is_last=False)

    def two_blocks(i, carry):
        block_ops(1 + 2 * i, 1, False, False)
        block_ops(2 + 2 * i, 0, False, False)
        return carry
    lax.fori_loop(0, (NBLK - 2) // 2, two_blocks, 0)

    block_ops(NBLK - 1, (NBLK - 1) % 2, is_first=False, is_last=True)
    wsc((NBLK - 1) % 2, BI - 1, 1)

    plsc.subcore_barrier()

    @pl.when(s < 15)
    def _():
        pltpu.sync_copy(acc.at[pl.ds(base, NR_MAIN)],
                        out_hbm.at[c, pl.ds(base, NR_MAIN)])

    @pl.when(s == 15)
    def _():
        pltpu.sync_copy(acc.at[pl.ds(15 * NR_MAIN, NR_TAIL)],
                        out_hbm.at[c, pl.ds(15 * NR_MAIN, NR_TAIL)])


RB = 1024
GRID = 10


def _dis_block(pdeg_ref, i):
    deg = pdeg_ref[0, pl.ds(i * RB, RB)] + pdeg_ref[1, pl.ds(i * RB, RB)] + 1.0
    return lax.rsqrt(deg)


def _mm1_body(pdeg_ref, x_ref, w_ref, o_ref):
    i = pl.program_id(0)
    dis = _dis_block(pdeg_ref, i)
    o_ref[...] = jnp.dot(x_ref[...], w_ref[...],
                         preferred_element_type=jnp.float32) * dis[:, None]


def _mid_body(pdeg_ref, p_ref, ht_ref, w_ref, b_ref, o_ref):
    i = pl.program_id(0)
    dis = _dis_block(pdeg_ref, i)
    agg = p_ref[0] + p_ref[1] + ht_ref[...]
    h1 = jnp.maximum(agg * dis[:, None] + b_ref[...][None, :], 0.0)
    o_ref[...] = jnp.dot(h1, w_ref[...],
                         preferred_element_type=jnp.float32) * dis[:, None]


def _fin_body(pdeg_ref, p_ref, ht_ref, b_ref, o_ref):
    i = pl.program_id(0)
    dis = _dis_block(pdeg_ref, i)
    agg = p_ref[0] + p_ref[1] + ht_ref[...]
    o_ref[...] = jax.nn.sigmoid(agg * dis[:, None] + b_ref[...][None, :])


_full_pdeg = pl.BlockSpec((NC, NP_DEG), lambda i: (0, 0))
_rows = pl.BlockSpec((RB, D), lambda i: (i, 0))
_part = pl.BlockSpec((NC, RB, D), lambda i: (0, i, 0))
_wspec = pl.BlockSpec((D, D), lambda i: (0, 0))
_bspec = pl.BlockSpec((D,), lambda i: (0,))
_out_sds = jax.ShapeDtypeStruct((N, D), jnp.float32)

_mm1 = pl.pallas_call(
    _mm1_body, grid=(GRID,),
    in_specs=[_full_pdeg, _rows, _wspec],
    out_specs=_rows, out_shape=_out_sds)

_mid = pl.pallas_call(
    _mid_body, grid=(GRID,),
    in_specs=[_full_pdeg, _part, _rows, _wspec, _bspec],
    out_specs=_rows, out_shape=_out_sds)

_fin = pl.pallas_call(
    _fin_body, grid=(GRID,),
    in_specs=[_full_pdeg, _part, _rows, _bspec],
    out_specs=_rows, out_shape=_out_sds)


def kernel(x, edge_index, edge_weight, W1, b1, W2, b2):
    pad = EP - E
    ei = edge_index.astype(jnp.int32)
    spread = jnp.arange(pad, dtype=jnp.int32) % N
    row = jnp.concatenate([ei[0], spread]).reshape(EP // CH, CH)
    col = jnp.concatenate([ei[1], spread]).reshape(EP // CH, CH)
    ew = jnp.concatenate([edge_weight, jnp.zeros((pad,), jnp.float32)]).reshape(EP // CH, CH)

    pdeg = _deg_kernel(col, ew)
    h1t = _mm1(pdeg, x, W1)
    p1 = _agg_kernel(h1t, row, col, ew)
    h2t = _mid(pdeg, p1, h1t, W2, b1)
    p2 = _agg_kernel(h2t, row, col, ew)
    return _fin(pdeg, p2, h2t, b2)


# speedup vs baseline: 27.7603x; 1.0080x over previous
import functools

import jax
import jax.numpy as jnp
from jax import lax
from jax.experimental import pallas as pl
from jax.experimental.pallas import tpu as pltpu
from jax.experimental.pallas import tpu_sc as plsc

N = 10000
E = 320000
D = 128

NC = 2
NS = 16
NW = NC * NS
CH = 128
EP = 327680
NCH = EP // CH // NW
BI = 8
NBLK = NCH // BI

NR_MAIN = 632
NR_TAIL = N - 15 * NR_MAIN

NP_DEG = 10240

_mesh = plsc.VectorSubcoreMesh(core_axis_name="c", subcore_axis_name="s")

_GATHER_DNUMS = lax.GatherDimensionNumbers(
    offset_dims=(), collapsed_slice_dims=(0,), start_index_map=(0,))


def _bcast_lane(v16, i):
    idx = jnp.full((16, 1), i, jnp.int32)
    return lax.gather(v16, idx, _GATHER_DNUMS, slice_sizes=(1,),
                      mode=lax.GatherScatterMode.PROMISE_IN_BOUNDS)


@functools.partial(
    pl.kernel, mesh=_mesh,
    out_type=jax.ShapeDtypeStruct((NC, NP_DEG), jnp.float32),
    scratch_types=[
        pltpu.VMEM((BI, CH), jnp.int32),
        pltpu.VMEM((BI, CH), jnp.float32),
        pltpu.VMEM((640,), jnp.float32),
        pltpu.VMEM_SHARED((NP_DEG,), jnp.float32),
        pltpu.SemaphoreType.DMA,
    ])
def _deg_kernel(col_hbm, ew_hbm, out_hbm, col_v, ew_v, zb, acc, dsem):
    c = lax.axis_index("c")
    s = lax.axis_index("s")
    wid = c * NS + s

    def zloop(i, carry):
        zb[pl.ds(i * 16, 16)] = jnp.zeros((16,), jnp.float32)
        return carry
    lax.fori_loop(0, 40, zloop, 0)

    pltpu.sync_copy(zb, acc.at[pl.ds(s * 640, 640)])

    plsc.subcore_barrier()

    def block(b, carry):
        r0 = wid * NCH + b * BI
        pltpu.sync_copy(col_hbm.at[pl.ds(r0, BI)], col_v)
        pltpu.sync_copy(ew_hbm.at[pl.ds(r0, BI)], ew_v)

        for k in range(BI):
            pltpu.async_copy(ew_v.at[k], acc.at[col_v.at[k]], dsem, add=True)
        for k in range(BI):
            pltpu.make_async_copy(ew_v.at[k], acc.at[col_v.at[k]], dsem).wait()
        return carry
    lax.fori_loop(0, NBLK, block, 0)

    plsc.subcore_barrier()
    pltpu.sync_copy(acc.at[pl.ds(s * 640, 640)], out_hbm.at[c, pl.ds(s * 640, 640)])


@functools.partial(
    pl.kernel, mesh=_mesh,
    out_type=jax.ShapeDtypeStruct((NC, N, D), jnp.float32),
    scratch_types=[
        pltpu.VMEM((2, BI, CH), jnp.int32),
        pltpu.VMEM((2, BI, CH), jnp.int32),
        pltpu.VMEM((2, BI, CH), jnp.float32),
        pltpu.VMEM((CH, D), jnp.float32),
        pltpu.VMEM((CH, D), jnp.float32),
        pltpu.VMEM_SHARED((N, D), jnp.float32),
        pltpu.SemaphoreType.DMA,
        pltpu.SemaphoreType.DMA,
        pltpu.SemaphoreType.DMA,
        pltpu.SemaphoreType.DMA,
        pltpu.SemaphoreType.DMA,
        pltpu.SemaphoreType.DMA,
    ])
def _agg_kernel(h_hbm, row_hbm, col_hbm, ew_hbm, out_hbm,
                rowb, colb, ewb, m0, m1, acc, g0, g1, s0, s1, t0, t1):
    c = lax.axis_index("c")
    s = lax.axis_index("s")
    wid = c * NS + s
    msgs = (m0, m1)
    gsem = (g0, g1)
    ssem = (s0, s1)
    tsem = (t0, t1)
    r0 = wid * NCH

    def stage(j, p, sync=False):
        copy = pltpu.sync_copy if sync else (
            lambda src, dst: pltpu.async_copy(src, dst, tsem[p]))
        copy(row_hbm.at[pl.ds(r0 + j * BI, BI)], rowb.at[p])
        copy(col_hbm.at[pl.ds(r0 + j * BI, BI)], colb.at[p])
        copy(ew_hbm.at[pl.ds(r0 + j * BI, BI)], ewb.at[p])

    def wait_stage(j, p):
        pltpu.make_async_copy(row_hbm.at[pl.ds(r0 + j * BI, BI)], rowb.at[p], tsem[p]).wait()
        pltpu.make_async_copy(col_hbm.at[pl.ds(r0 + j * BI, BI)], colb.at[p], tsem[p]).wait()
        pltpu.make_async_copy(ew_hbm.at[pl.ds(r0 + j * BI, BI)], ewb.at[p], tsem[p]).wait()

    def ig(p, kk, b):
        pltpu.async_copy(h_hbm.at[rowb.at[p, kk]], msgs[b], gsem[b])

    def wg(p, kk, b):
        pltpu.make_async_copy(h_hbm.at[rowb.at[p, kk]], msgs[b], gsem[b]).wait()

    def isc(p, kk, b):
        pltpu.async_copy(msgs[b], acc.at[colb.at[p, kk]], ssem[b], add=True)

    def wsc(p, kk, b):
        pltpu.make_async_copy(msgs[b], acc.at[colb.at[p, kk]], ssem[b]).wait()

    def scale(p, kk, b):
        mb = msgs[b]

        def scale16(g, carry):
            ew16 = ewb[p, kk, pl.ds(g * 16, 16)]
            for i in range(16):
                nb = _bcast_lane(ew16, i)
                e = g * 16 + i
                for jj in range(D // 16):
                    mb[e, pl.ds(jj * 16, 16)] = mb[e, pl.ds(jj * 16, 16)] * nb
            return carry
        lax.fori_loop(0, CH // 16, scale16, 0)

    def zrow(i, carry):
        for j in range(D // 16):
            m0[i, pl.ds(j * 16, 16)] = jnp.zeros((16,), jnp.float32)
        return carry
    lax.fori_loop(0, CH, zrow, 0)

    base = s * NR_MAIN

    @pl.when(s < 15)
    def _():
        for q in range(NR_MAIN // CH):
            pltpu.sync_copy(m0, acc.at[pl.ds(base + q * CH, CH)])
        rem = NR_MAIN - (NR_MAIN // CH) * CH
        pltpu.sync_copy(m0.at[pl.ds(0, rem)],
                        acc.at[pl.ds(base + NR_MAIN - rem, rem)])

    @pl.when(s == 15)
    def _():
        for q in range(NR_TAIL // CH):
            pltpu.sync_copy(m0, acc.at[pl.ds(15 * NR_MAIN + q * CH, CH)])
        rem = NR_TAIL - (NR_TAIL // CH) * CH
        pltpu.sync_copy(m0.at[pl.ds(0, rem)],
                        acc.at[pl.ds(15 * NR_MAIN + NR_TAIL - rem, rem)])

    plsc.subcore_barrier()

    def block_ops(j, p, is_first, is_last):
        for kk in range(BI):
            b = kk % 2
            if kk == 0:
                if not is_first:
                    wsc(p ^ 1, BI - 1, 1)
                if not is_last:
                    stage(j + 1, p ^ 1)
            else:
                wsc(p, kk - 1, (kk - 1) % 2)
            if kk == BI - 1:
                if not is_last:
                    wait_stage(j + 1, p ^ 1)
                    ig(p ^ 1, 0, 0)
            else:
                ig(p, kk + 1, (kk + 1) % 2)
            wg(p, kk, b)
            scale(p, kk, b)
            isc(p, kk, b)

    stage(0, 0, sync=True)
    ig(0, 0, 0)
    block_ops(0, 0, is_first=True, is_last=False)

    def two_blocks(i, carry):
        block_ops(1 + 2 * i, 1, False, False)
        block_ops(2 + 2 * i, 0, False, False)
        return carry
    lax.fori_loop(0, (NBLK - 2) // 2, two_blocks, 0)

    block_ops(NBLK - 1, (NBLK - 1) % 2, is_first=False, is_last=True)
    wsc((NBLK - 1) % 2, BI - 1, 1)

    plsc.subcore_barrier()

    @pl.when(s < 15)
    def _():
        pltpu.sync_copy(acc.at[pl.ds(base, NR_MAIN)],
                        out_hbm.at[c, pl.ds(base, NR_MAIN)])

    @pl.when(s == 15)
    def _():
        pltpu.sync_copy(acc.at[pl.ds(15 * NR_MAIN, NR_TAIL)],
                        out_hbm.at[c, pl.ds(15 * NR_MAIN, NR_TAIL)])


RB = 1024
GRID = 10


def _dis_block(pdeg_ref, i):
    deg = pdeg_ref[0, pl.ds(i * RB, RB)] + pdeg_ref[1, pl.ds(i * RB, RB)] + 1.0
    return lax.rsqrt(deg)


def _mm1_body(pdeg_ref, x_ref, w_ref, o_ref):
    i = pl.program_id(0)
    dis = _dis_block(pdeg_ref, i)
    o_ref[...] = jnp.dot(x_ref[...], w_ref[...],
                         preferred_element_type=jnp.float32) * dis[:, None]


def _mid_body(pdeg_ref, p_ref, ht_ref, w_ref, b_ref, o_ref):
    i = pl.program_id(0)
    dis = _dis_block(pdeg_ref, i)
    agg = p_ref[0] + p_ref[1] + ht_ref[...]
    h1 = jnp.maximum(agg * dis[:, None] + b_ref[...][None, :], 0.0)
    o_ref[...] = jnp.dot(h1, w_ref[...],
                         preferred_element_type=jnp.float32) * dis[:, None]


def _fin_body(pdeg_ref, p_ref, ht_ref, b_ref, o_ref):
    i = pl.program_id(0)
    dis = _dis_block(pdeg_ref, i)
    agg = p_ref[0] + p_ref[1] + ht_ref[...]
    o_ref[...] = jax.nn.sigmoid(agg * dis[:, None] + b_ref[...][None, :])


_full_pdeg = pl.BlockSpec((NC, NP_DEG), lambda i: (0, 0))
_rows = pl.BlockSpec((RB, D), lambda i: (i, 0))
_part = pl.BlockSpec((NC, RB, D), lambda i: (0, i, 0))
_wspec = pl.BlockSpec((D, D), lambda i: (0, 0))
_bspec = pl.BlockSpec((D,), lambda i: (0,))
_out_sds = jax.ShapeDtypeStruct((N, D), jnp.float32)

_mm1 = pl.pallas_call(
    _mm1_body, grid=(GRID,),
    in_specs=[_full_pdeg, _rows, _wspec],
    out_specs=_rows, out_shape=_out_sds)

_mid = pl.pallas_call(
    _mid_body, grid=(GRID,),
    in_specs=[_full_pdeg, _part, _rows, _wspec, _bspec],
    out_specs=_rows, out_shape=_out_sds)

_fin = pl.pallas_call(
    _fin_body, grid=(GRID,),
    in_specs=[_full_pdeg, _part, _rows, _bspec],
    out_specs=_rows, out_shape=_out_sds)


def kernel(x, edge_index, edge_weight, W1, b1, W2, b2):
    pad = EP - E
    ei = edge_index.astype(jnp.int32)
    spread = jnp.arange(pad, dtype=jnp.int32) % N
    row = jnp.concatenate([ei[0], spread]).reshape(EP // CH, CH)
    col = jnp.concatenate([ei[1], spread]).reshape(EP // CH, CH)
    ew = jnp.concatenate([edge_weight, jnp.zeros((pad,), jnp.float32)]).reshape(EP // CH, CH)

    pdeg = _deg_kernel(col, ew)
    h1t = _mm1(pdeg, x, W1)
    p1 = _agg_kernel(h1t, row, col, ew)
    h2t = _mid(pdeg, p1, h1t, W2, b1)
    p2 = _agg_kernel(h2t, row, col, ew)
    return _fin(pdeg, p2, h2t, b2)


# speedup vs baseline: 27.8530x; 1.0033x over previous
import functools

import jax
import jax.numpy as jnp
from jax import lax
from jax.experimental import pallas as pl
from jax.experimental.pallas import tpu as pltpu
from jax.experimental.pallas import tpu_sc as plsc

N = 10000
E = 320000
D = 128

NC = 2
NS = 16
NW = NC * NS
CH = 128
EP = 327680
NCH = EP // CH // NW
BI = 8
NBLK = NCH // BI

NR_MAIN = 632
NR_TAIL = N - 15 * NR_MAIN

NP_DEG = 10240

_mesh = plsc.VectorSubcoreMesh(core_axis_name="c", subcore_axis_name="s")

_GATHER_DNUMS = lax.GatherDimensionNumbers(
    offset_dims=(), collapsed_slice_dims=(0,), start_index_map=(0,))


def _bcast_lane(v16, i):
    idx = jnp.full((16, 1), i, jnp.int32)
    return lax.gather(v16, idx, _GATHER_DNUMS, slice_sizes=(1,),
                      mode=lax.GatherScatterMode.PROMISE_IN_BOUNDS)


@functools.partial(
    pl.kernel, mesh=_mesh,
    out_type=jax.ShapeDtypeStruct((NC, NP_DEG), jnp.float32),
    scratch_types=[
        pltpu.VMEM((BI, CH), jnp.int32),
        pltpu.VMEM((BI, CH), jnp.float32),
        pltpu.VMEM((640,), jnp.float32),
        pltpu.VMEM_SHARED((NP_DEG,), jnp.float32),
        pltpu.SemaphoreType.DMA,
    ])
def _deg_kernel(col_hbm, ew_hbm, out_hbm, col_v, ew_v, zb, acc, dsem):
    c = lax.axis_index("c")
    s = lax.axis_index("s")
    wid = c * NS + s

    def zloop(i, carry):
        zb[pl.ds(i * 16, 16)] = jnp.zeros((16,), jnp.float32)
        return carry
    lax.fori_loop(0, 40, zloop, 0)

    pltpu.sync_copy(zb, acc.at[pl.ds(s * 640, 640)])

    plsc.subcore_barrier()

    def block(b, carry):
        r0 = wid * NCH + b * BI
        pltpu.sync_copy(col_hbm.at[pl.ds(r0, BI)], col_v)
        pltpu.sync_copy(ew_hbm.at[pl.ds(r0, BI)], ew_v)

        for k in range(BI):
            pltpu.async_copy(ew_v.at[k], acc.at[col_v.at[k]], dsem, add=True)
        for k in range(BI):
            pltpu.make_async_copy(ew_v.at[k], acc.at[col_v.at[k]], dsem).wait()
        return carry
    lax.fori_loop(0, NBLK, block, 0)

    plsc.subcore_barrier()
    pltpu.sync_copy(acc.at[pl.ds(s * 640, 640)], out_hbm.at[c, pl.ds(s * 640, 640)])


@functools.partial(
    pl.kernel, mesh=_mesh,
    out_type=jax.ShapeDtypeStruct((NC, N, D), jnp.float32),
    scratch_types=[
        pltpu.VMEM((2, BI, CH), jnp.int32),
        pltpu.VMEM((2, BI, CH), jnp.int32),
        pltpu.VMEM((2, BI, CH), jnp.float32),
        pltpu.VMEM((CH, D), jnp.float32),
        pltpu.VMEM((CH, D), jnp.float32),
        pltpu.VMEM_SHARED((N, D), jnp.float32),
        pltpu.SemaphoreType.DMA,
        pltpu.SemaphoreType.DMA,
        pltpu.SemaphoreType.DMA,
        pltpu.SemaphoreType.DMA,
        pltpu.SemaphoreType.DMA,
        pltpu.SemaphoreType.DMA,
    ])
def _agg_kernel(h_hbm, row_hbm, col_hbm, ew_hbm, out_hbm,
                rowb, colb, ewb, m0, m1, acc, g0, g1, s0, s1, t0, t1):
    c = lax.axis_index("c")
    s = lax.axis_index("s")
    wid = c * NS + s
    msgs = (m0, m1)
    gsem = (g0, g1)
    ssem = (s0, s1)
    tsem = (t0, t1)
    r0 = wid * NCH

    def stage(j, p, sync=False):
        copy = pltpu.sync_copy if sync else (
            lambda src, dst: pltpu.async_copy(src, dst, tsem[p]))
        copy(row_hbm.at[pl.ds(r0 + j * BI, BI)], rowb.at[p])
        copy(col_hbm.at[pl.ds(r0 + j * BI, BI)], colb.at[p])
        copy(ew_hbm.at[pl.ds(r0 + j * BI, BI)], ewb.at[p])

    def wait_stage(j, p):
        pltpu.make_async_copy(row_hbm.at[pl.ds(r0 + j * BI, BI)], rowb.at[p], tsem[p]).wait()
        pltpu.make_async_copy(col_hbm.at[pl.ds(r0 + j * BI, BI)], colb.at[p], tsem[p]).wait()
        pltpu.make_async_copy(ew_hbm.at[pl.ds(r0 + j * BI, BI)], ewb.at[p], tsem[p]).wait()

    def ig(p, kk, b):
        pltpu.async_copy(h_hbm.at[rowb.at[p, kk, pl.ds(0, CH // 2)]],
                         msgs[b].at[pl.ds(0, CH // 2)], gsem[b])
        pltpu.async_copy(h_hbm.at[rowb.at[p, kk, pl.ds(CH // 2, CH // 2)]],
                         msgs[b].at[pl.ds(CH // 2, CH // 2)], gsem[b])

    def wg(p, kk, b):
        pltpu.make_async_copy(h_hbm.at[rowb.at[p, kk, pl.ds(0, CH // 2)]],
                              msgs[b].at[pl.ds(0, CH // 2)], gsem[b]).wait()
        pltpu.make_async_copy(h_hbm.at[rowb.at[p, kk, pl.ds(CH // 2, CH // 2)]],
                              msgs[b].at[pl.ds(CH // 2, CH // 2)], gsem[b]).wait()

    def isc(p, kk, b):
        pltpu.async_copy(msgs[b], acc.at[colb.at[p, kk]], ssem[b], add=True)

    def wsc(p, kk, b):
        pltpu.make_async_copy(msgs[b], acc.at[colb.at[p, kk]], ssem[b]).wait()

    def scale(p, kk, b):
        mb = msgs[b]

        def scale16(g, carry):
            ew16 = ewb[p, kk, pl.ds(g * 16, 16)]
            for i in range(16):
                nb = _bcast_lane(ew16, i)
                e = g * 16 + i
                for jj in range(D // 16):
                    mb[e, pl.ds(jj * 16, 16)] = mb[e, pl.ds(jj * 16, 16)] * nb
            return carry
        lax.fori_loop(0, CH // 16, scale16, 0)

    def zrow(i, carry):
        for j in range(D // 16):
            m0[i, pl.ds(j * 16, 16)] = jnp.zeros((16,), jnp.float32)
        return carry
    lax.fori_loop(0, CH, zrow, 0)

    base = s * NR_MAIN

    @pl.when(s < 15)
    def _():
        for q in range(NR_MAIN // CH):
            pltpu.sync_copy(m0, acc.at[pl.ds(base + q * CH, CH)])
        rem = NR_MAIN - (NR_MAIN // CH) * CH
        pltpu.sync_copy(m0.at[pl.ds(0, rem)],
                        acc.at[pl.ds(base + NR_MAIN - rem, rem)])

    @pl.when(s == 15)
    def _():
        for q in range(NR_TAIL // CH):
            pltpu.sync_copy(m0, acc.at[pl.ds(15 * NR_MAIN + q * CH, CH)])
        rem = NR_TAIL - (NR_TAIL // CH) * CH
        pltpu.sync_copy(m0.at[pl.ds(0, rem)],
                        acc.at[pl.ds(15 * NR_MAIN + NR_TAIL - rem, rem)])

    plsc.subcore_barrier()

    def block_ops(j, p, is_first, is_last):
        for kk in range(BI):
            b = kk % 2
            if kk == 0:
                if not is_first:
                    wsc(p ^ 1, BI - 1, 1)
                if not is_last:
                    stage(j + 1, p ^ 1)
            else:
                wsc(p, kk - 1, (kk - 1) % 2)
            if kk == BI - 1:
                if not is_last:
                    wait_stage(j + 1, p ^ 1)
                    ig(p ^ 1, 0, 0)
            else:
                ig(p, kk + 1, (kk + 1) % 2)
            wg(p, kk, b)
            scale(p, kk, b)
            isc(p, kk, b)

    stage(0, 0, sync=True)
    ig(0, 0, 0)
    block_ops(0, 0, is_first=True, is_last=False)

    def two_blocks(i, carry):
        block_ops(1 + 2 * i, 1, False, False)
        block_ops(2 + 2 * i, 0, False, False)
        return carry
    lax.fori_loop(0, (NBLK - 2) // 2, two_blocks, 0)

    block_ops(NBLK - 1, (NBLK - 1) % 2, is_first=False, is_last=True)
    wsc((NBLK - 1) % 2, BI - 1, 1)

    plsc.subcore_barrier()

    @pl.when(s < 15)
    def _():
        pltpu.sync_copy(acc.at[pl.ds(base, NR_MAIN)],
                        out_hbm.at[c, pl.ds(base, NR_MAIN)])

    @pl.when(s == 15)
    def _():
        pltpu.sync_copy(acc.at[pl.ds(15 * NR_MAIN, NR_TAIL)],
                        out_hbm.at[c, pl.ds(15 * NR_MAIN, NR_TAIL)])


RB = 1024
GRID = 10


def _dis_block(pdeg_ref, i):
    deg = pdeg_ref[0, pl.ds(i * RB, RB)] + pdeg_ref[1, pl.ds(i * RB, RB)] + 1.0
    return lax.rsqrt(deg)


def _mm1_body(pdeg_ref, x_ref, w_ref, o_ref):
    i = pl.program_id(0)
    dis = _dis_block(pdeg_ref, i)
    o_ref[...] = jnp.dot(x_ref[...], w_ref[...],
                         preferred_element_type=jnp.float32) * dis[:, None]


def _mid_body(pdeg_ref, p_ref, ht_ref, w_ref, b_ref, o_ref):
    i = pl.program_id(0)
    dis = _dis_block(pdeg_ref, i)
    agg = p_ref[0] + p_ref[1] + ht_ref[...]
    h1 = jnp.maximum(agg * dis[:, None] + b_ref[...][None, :], 0.0)
    o_ref[...] = jnp.dot(h1, w_ref[...],
                         preferred_element_type=jnp.float32) * dis[:, None]


def _fin_body(pdeg_ref, p_ref, ht_ref, b_ref, o_ref):
    i = pl.program_id(0)
    dis = _dis_block(pdeg_ref, i)
    agg = p_ref[0] + p_ref[1] + ht_ref[...]
    o_ref[...] = jax.nn.sigmoid(agg * dis[:, None] + b_ref[...][None, :])


_full_pdeg = pl.BlockSpec((NC, NP_DEG), lambda i: (0, 0))
_rows = pl.BlockSpec((RB, D), lambda i: (i, 0))
_part = pl.BlockSpec((NC, RB, D), lambda i: (0, i, 0))
_wspec = pl.BlockSpec((D, D), lambda i: (0, 0))
_bspec = pl.BlockSpec((D,), lambda i: (0,))
_out_sds = jax.ShapeDtypeStruct((N, D), jnp.float32)

_mm1 = pl.pallas_call(
    _mm1_body, grid=(GRID,),
    in_specs=[_full_pdeg, _rows, _wspec],
    out_specs=_rows, out_shape=_out_sds)

_mid = pl.pallas_call(
    _mid_body, grid=(GRID,),
    in_specs=[_full_pdeg, _part, _rows, _wspec, _bspec],
    out_specs=_rows, out_shape=_out_sds)

_fin = pl.pallas_call(
    _fin_body, grid=(GRID,),
    in_specs=[_full_pdeg, _part, _rows, _bspec],
    out_specs=_rows, out_shape=_out_sds)


def kernel(x, edge_index, edge_weight, W1, b1, W2, b2):
    pad = EP - E
    ei = edge_index.astype(jnp.int32)
    spread = jnp.arange(pad, dtype=jnp.int32) % N
    row = jnp.concatenate([ei[0], spread]).reshape(EP // CH, CH)
    col = jnp.concatenate([ei[1], spread]).reshape(EP // CH, CH)
    ew = jnp.concatenate([edge_weight, jnp.zeros((pad,), jnp.float32)]).reshape(EP // CH, CH)

    pdeg = _deg_kernel(col, ew)
    h1t = _mm1(pdeg, x, W1)
    p1 = _agg_kernel(h1t, row, col, ew)
    h2t = _mid(pdeg, p1, h1t, W2, b1)
    p2 = _agg_kernel(h2t, row, col, ew)
    return _fin(pdeg, p2, h2t, b2)
